# baseline (speedup 1.0000x reference)
import jax
import jax.numpy as jnp
from jax.experimental import pallas as pl

B, C, H, W = 8, 96, 224, 224
HB = 56


def _body(x_ref, o_ref):
    v = x_ref[0]
    m = jnp.max(v, axis=0, keepdims=True)
    e = jnp.exp(v - m)
    s = jnp.sum(e, axis=0, keepdims=True)
    y = v - (m + jnp.log(s))
    ts = [y[:, i * W:(i + 1) * W].T for i in range(HB)]
    o_ref[0] = jnp.concatenate(ts, axis=1)


def kernel(x):
    xf = x.reshape(B, C, H * W)
    out = pl.pallas_call(
        _body,
        grid=(B, H // HB),
        in_specs=[pl.BlockSpec((1, C, HB * W), lambda b, h: (b, 0, h))],
        out_specs=pl.BlockSpec((1, W, HB * C), lambda b, h: (b, 0, h)),
        out_shape=jax.ShapeDtypeStruct((B, W, H * C), x.dtype),
    )(xf)
    return out.reshape(-1, C)


# speedup vs baseline: 1.5948x; 1.5948x over previous
import jax
import jax.numpy as jnp
from jax.experimental import pallas as pl

B, C, H, W = 8, 96, 224, 224
HB = 112


def _body(x_ref, o_ref):
    v = x_ref[0]
    m = jnp.max(v, axis=0, keepdims=True)
    e = jnp.exp(v - m)
    s = jnp.sum(e, axis=0, keepdims=True)
    y = v - (m + jnp.log(s))
    for i in range(HB):
        o_ref[0, :, i, :] = y[:, i, :].T


def kernel(x):
    out = pl.pallas_call(
        _body,
        grid=(B, H // HB),
        in_specs=[pl.BlockSpec((1, C, HB, W), lambda b, h: (b, 0, h, 0))],
        out_specs=pl.BlockSpec((1, W, HB, C), lambda b, h: (b, 0, h, 0)),
        out_shape=jax.ShapeDtypeStruct((B, W, H, C), x.dtype),
    )(x)
    return out.reshape(-1, C)


# speedup vs baseline: 1.6258x; 1.0195x over previous
import jax
import jax.numpy as jnp
from jax.experimental import pallas as pl

B, C, H, W = 8, 96, 224, 224
HB = 56


def _body(x_ref, o_ref):
    v = x_ref[0]
    m = jnp.max(v, axis=0, keepdims=True)
    e = jnp.exp(v - m)
    s = jnp.sum(e, axis=0, keepdims=True)
    y = v - (m + jnp.log(s))
    for i in range(HB):
        o_ref[0, :, i, :] = y[:, i, :].T


def kernel(x):
    out = pl.pallas_call(
        _body,
        grid=(B, H // HB),
        in_specs=[pl.BlockSpec((1, C, HB, W), lambda b, h: (b, 0, h, 0))],
        out_specs=pl.BlockSpec((1, W, HB, C), lambda b, h: (b, 0, h, 0)),
        out_shape=jax.ShapeDtypeStruct((B, W, H, C), x.dtype),
    )(x)
    return out.reshape(-1, C)
